# baseline (speedup 1.0000x reference)
import functools

import jax
import jax.numpy as jnp
from jax import lax
from jax.experimental import pallas as pl
from jax.experimental.pallas import tpu as pltpu
from jax.experimental.pallas import tpu_sc as plsc

N_BATCH = 1024
DIM = 32
N_STATES = 256
RADIUS = 8
TEMP = 2.0
WS = 2 * RADIUS + 1

NUM_WORKERS = 32
ROWS_PER_WORKER = N_BATCH // NUM_WORKERS
GROUPS = ROWS_PER_WORKER // 16

LN2 = 0.6931471805599453
_LN_POLY = (0.030449, -0.28382685, 1.11609003, -2.44002976, 3.5140873,
            -1.93675974)


def _vlog(s):
    bits = plsc.bitcast(s, jnp.int32)
    e = (bits >> 23) - 127
    mant = plsc.bitcast((bits & 0x007FFFFF) | 0x3F800000, jnp.float32)
    p = jnp.full((16,), _LN_POLY[0], jnp.float32)
    for coef in _LN_POLY[1:]:
        p = p * mant + coef
    return e.astype(jnp.float32) * LN2 + p


def _sc_body(x_hbm, s_hbm, w_hbm, ss_hbm, out_hbm,
             xv, sv, wv, ssv, tlin, tlse, pfx, ov,
             sem_x, sem_s, sem_w, sem_ss):
    wid = lax.axis_index("s") * 2 + lax.axis_index("c")
    base = wid * ROWS_PER_WORKER
    fbase = base * DIM
    cp_x = pltpu.async_copy(
        x_hbm.at[pl.ds(fbase, ROWS_PER_WORKER * DIM)], xv, sem_x)
    cp_s = pltpu.async_copy(
        s_hbm.at[pl.ds(fbase, ROWS_PER_WORKER * DIM)], sv, sem_s)
    cp_w = pltpu.async_copy(w_hbm, wv, sem_w)
    cp_ss = pltpu.async_copy(ss_hbm.at[pl.ds(0, 32)], ssv, sem_ss)
    cp_w.wait()
    cp_ss.wait()

    iota16 = lax.iota(jnp.int32, 16)
    s0 = plsc.load_gather(ssv, [iota16])
    s1 = plsc.load_gather(ssv, [iota16 + 1])
    a = (s1 - s0) * (1.0 / TEMP)
    for h in range(DIM // 16):
        sl = pl.ds(h * 16, 16)
        w_h = wv[sl]

        def k_step(k, run, w_h=w_h, sl=sl):
            arg = w_h * (a * (k.astype(jnp.float32) - RADIUS))
            tlin[k, sl] = arg
            run = run + jnp.exp(arg)
            pfx[k, sl] = run
            return run

        lax.fori_loop(0, WS, k_step, jnp.zeros((16,), jnp.float32))

        def c_step(c, carry, sl=sl):
            hi = jnp.minimum(c + RADIUS, WS - 1)
            lo = jnp.maximum(c - RADIUS - 1, 0)
            ssum = pfx[hi, sl]
            ssum = ssum - jnp.where(c >= RADIUS + 1, pfx[lo, sl], 0.0)
            tlse[c, sl] = _vlog(ssum)
            return carry

        lax.fori_loop(0, WS, c_step, jnp.float32(0.0))

    cp_x.wait()
    cp_s.wait()
    rvecs = tuple(iota16 + (g * 16) for g in range(GROUPS))

    def dim_step(i, accs):
        ifull = jnp.full((16,), i, jnp.int32)
        new_accs = []
        for g in range(GROUPS):
            flat = rvecs[g] * DIM + ifull
            x = plsc.load_gather(xv, [flat])
            s = plsc.load_gather(sv, [flat])
            j = jnp.clip(s - x + RADIUS, 0, WS - 1)
            c = RADIUS + jnp.maximum(0, RADIUS - x) \
                - jnp.maximum(0, x - (N_STATES - 1 - RADIUS))
            c = jnp.clip(c, 0, WS - 1)
            lin = plsc.load_gather(tlin, [j, ifull])
            lse = plsc.load_gather(tlse, [c, ifull])
            new_accs.append(accs[g] + (lin - lse))
        return tuple(new_accs)

    accs = lax.fori_loop(0, DIM, dim_step,
                         tuple(jnp.zeros((16,), jnp.float32)
                               for _ in range(GROUPS)))
    for g in range(GROUPS):
        ov[pl.ds(g * 16, 16)] = accs[g]
    pltpu.sync_copy(ov, out_hbm.at[pl.ds(base, ROWS_PER_WORKER)])


@functools.partial(
    pl.kernel,
    mesh=plsc.VectorSubcoreMesh(core_axis_name="c", subcore_axis_name="s"),
    out_type=jax.ShapeDtypeStruct((N_BATCH,), jnp.float32),
    compiler_params=pltpu.CompilerParams(needs_layout_passes=False,
                                         skip_device_barrier=True),
    scratch_types=[
        pltpu.VMEM((ROWS_PER_WORKER * DIM,), jnp.int32),
        pltpu.VMEM((ROWS_PER_WORKER * DIM,), jnp.int32),
        pltpu.VMEM((DIM,), jnp.float32),
        pltpu.VMEM((32,), jnp.float32),
        pltpu.VMEM((WS, DIM), jnp.float32),
        pltpu.VMEM((WS, DIM), jnp.float32),
        pltpu.VMEM((WS, DIM), jnp.float32),
        pltpu.VMEM((ROWS_PER_WORKER,), jnp.float32),
        pltpu.SemaphoreType.DMA,
        pltpu.SemaphoreType.DMA,
        pltpu.SemaphoreType.DMA,
        pltpu.SemaphoreType.DMA,
    ],
)
def _sc_fused(x, s, w, ss, out, xv, sv, wv, ssv, tlin, tlse, pfx, ov,
              sem_x, sem_s, sem_w, sem_ss):
    _sc_body(x, s, w, ss, out, xv, sv, wv, ssv, tlin, tlse, pfx, ov,
             sem_x, sem_s, sem_w, sem_ss)


def kernel(state_space, w, b, x_idx, s_idx):
    del b
    return _sc_fused(x_idx.astype(jnp.int32).reshape(-1),
                     s_idx.astype(jnp.int32).reshape(-1),
                     w.astype(jnp.float32), state_space.astype(jnp.float32))


# speedup vs baseline: 1.0169x; 1.0169x over previous
import functools

import jax
import jax.numpy as jnp
from jax import lax
from jax.experimental import pallas as pl
from jax.experimental.pallas import tpu as pltpu
from jax.experimental.pallas import tpu_sc as plsc

N_BATCH = 1024
DIM = 32
N_STATES = 256
RADIUS = 8
TEMP = 2.0
WS = 2 * RADIUS + 1

NUM_WORKERS = 32
ROWS_PER_WORKER = N_BATCH // NUM_WORKERS
GROUPS = ROWS_PER_WORKER // 16

LN2 = 0.6931471805599453
_LN_POLY = (0.030449, -0.28382685, 1.11609003, -2.44002976, 3.5140873,
            -1.93675974)


def _vlog(s):
    bits = plsc.bitcast(s, jnp.int32)
    e = (bits >> 23) - 127
    mant = plsc.bitcast((bits & 0x007FFFFF) | 0x3F800000, jnp.float32)
    p = jnp.full((16,), _LN_POLY[0], jnp.float32)
    for coef in _LN_POLY[1:]:
        p = p * mant + coef
    return e.astype(jnp.float32) * LN2 + p


def _sc_body(x_hbm, s_hbm, w_hbm, ss_hbm, out_hbm,
             xv, sv, wv, ssv, tlin, tlse, ov,
             sem_x, sem_s, sem_w, sem_ss):
    wid = lax.axis_index("s") * 2 + lax.axis_index("c")
    base = wid * ROWS_PER_WORKER
    cp_x = pltpu.async_copy(x_hbm.at[pl.ds(base, ROWS_PER_WORKER)], xv, sem_x)
    cp_s = pltpu.async_copy(s_hbm.at[pl.ds(base, ROWS_PER_WORKER)], sv, sem_s)
    cp_w = pltpu.async_copy(w_hbm, wv, sem_w)
    cp_ss = pltpu.async_copy(ss_hbm.at[pl.ds(0, 32)], ssv, sem_ss)
    cp_w.wait()
    cp_ss.wait()

    iota16 = lax.iota(jnp.int32, 16)
    s0 = plsc.load_gather(ssv, [iota16])
    s1 = plsc.load_gather(ssv, [iota16 + 1])
    a = (s1 - s0) * (1.0 / TEMP)
    for h in range(DIM // 16):
        sl = pl.ds(h * 16, 16)
        w_h = wv[sl]
        prefix = []
        run = None
        for k in range(WS):
            arg = w_h * (a * float(k - RADIUS))
            tlin[k, sl] = arg
            run = jnp.exp(arg) if run is None else run + jnp.exp(arg)
            prefix.append(run)
        for c in range(WS):
            hi = min(WS - 1, c + RADIUS)
            ssum = prefix[hi]
            if c - RADIUS - 1 >= 0:
                ssum = ssum - prefix[c - RADIUS - 1]
            tlse[c, sl] = _vlog(ssum)

    cp_x.wait()
    cp_s.wait()
    rvecs = tuple(iota16 + (g * 16) for g in range(GROUPS))

    def dim_step(i, accs):
        ifull = jnp.full((16,), i, jnp.int32)
        new_accs = []
        for g in range(GROUPS):
            x = plsc.load_gather(xv, [rvecs[g], ifull])
            s = plsc.load_gather(sv, [rvecs[g], ifull])
            j = jnp.clip(s - x + RADIUS, 0, WS - 1)
            c = RADIUS + jnp.maximum(0, RADIUS - x) \
                - jnp.maximum(0, x - (N_STATES - 1 - RADIUS))
            c = jnp.clip(c, 0, WS - 1)
            lin = plsc.load_gather(tlin, [j, ifull])
            lse = plsc.load_gather(tlse, [c, ifull])
            new_accs.append(accs[g] + (lin - lse))
        return tuple(new_accs)

    accs = lax.fori_loop(0, DIM, dim_step,
                         tuple(jnp.zeros((16,), jnp.float32)
                               for _ in range(GROUPS)))
    for g in range(GROUPS):
        ov[pl.ds(g * 16, 16)] = accs[g]
    pltpu.sync_copy(ov, out_hbm.at[pl.ds(base, ROWS_PER_WORKER)])


@functools.partial(
    pl.kernel,
    mesh=plsc.VectorSubcoreMesh(core_axis_name="c", subcore_axis_name="s"),
    out_type=jax.ShapeDtypeStruct((N_BATCH,), jnp.float32),
    compiler_params=pltpu.CompilerParams(needs_layout_passes=False,
                                         skip_device_barrier=True,
                                         use_tc_tiling_on_sc=True),
    scratch_types=[
        pltpu.VMEM((ROWS_PER_WORKER, DIM), jnp.int32),
        pltpu.VMEM((ROWS_PER_WORKER, DIM), jnp.int32),
        pltpu.VMEM((DIM,), jnp.float32),
        pltpu.VMEM((32,), jnp.float32),
        pltpu.VMEM((WS, DIM), jnp.float32),
        pltpu.VMEM((WS, DIM), jnp.float32),
        pltpu.VMEM((ROWS_PER_WORKER,), jnp.float32),
        pltpu.SemaphoreType.DMA,
        pltpu.SemaphoreType.DMA,
        pltpu.SemaphoreType.DMA,
        pltpu.SemaphoreType.DMA,
    ],
)
def _sc_fused(x, s, w, ss, out, xv, sv, wv, ssv, tlin, tlse, ov,
              sem_x, sem_s, sem_w, sem_ss):
    _sc_body(x, s, w, ss, out, xv, sv, wv, ssv, tlin, tlse, ov,
             sem_x, sem_s, sem_w, sem_ss)


def kernel(state_space, w, b, x_idx, s_idx):
    del b
    return _sc_fused(x_idx.astype(jnp.int32), s_idx.astype(jnp.int32),
                     w.astype(jnp.float32), state_space.astype(jnp.float32))
